# baseline (speedup 1.0000x reference)
import functools

import jax
import jax.numpy as jnp
from jax import lax
from jax.experimental import pallas as pl
from jax.experimental.pallas import tpu as pltpu
from jax.experimental.pallas import tpu_sc as plsc

_NC = 2
_NS = 16
_NW = _NC * _NS
_L = 16
_J = 10
_CHUNK = 128


@functools.cache
def _make_sc_kernel(B, N):
    b_per_w = B // _NW
    rows_per_j = b_per_w // _CHUNK
    n_rows = _J * rows_per_j
    q_per_row = _CHUNK // _L
    chunks = b_per_w // _L

    mesh = plsc.VectorSubcoreMesh(
        core_axis_name="c", subcore_axis_name="s",
        num_cores=_NC, num_subcores=_NS)

    @functools.partial(
        pl.kernel,
        out_type=jax.ShapeDtypeStruct((B,), jnp.float32),
        mesh=mesh,
        compiler_params=pltpu.CompilerParams(needs_layout_passes=False),
        scratch_types=[
            pltpu.VMEM((_L,), jnp.float32),
            pltpu.VMEM((_L,), jnp.float32),
            pltpu.VMEM((_L,), jnp.float32),
            pltpu.VMEM((_L,), jnp.int32),
            pltpu.VMEM((n_rows, _CHUNK), jnp.int32),
            pltpu.VMEM((n_rows, _CHUNK), jnp.float32),
            pltpu.VMEM((b_per_w,), jnp.float32),
            pltpu.SemaphoreType.DMA,
        ],
    )
    def sc_kernel(xflat, mean16, std16, vals16, idx16, out,
                  mean_v, std_v, vals_v, idx_v, idxbuf, gbuf, outbuf,
                  sem):
        wid = lax.axis_index("s") * _NC + lax.axis_index("c")
        base = wid * b_per_w

        pltpu.sync_copy(mean16, mean_v)
        pltpu.sync_copy(std16, std_v)
        pltpu.sync_copy(vals16, vals_v)
        pltpu.sync_copy(idx16, idx_v)

        lane = lax.iota(jnp.int32, _L)
        ramp = lane * N
        base_flat = base * N

        def lane_scalar(vec, j):
            return jnp.sum(jnp.where(lane == j, vec, jnp.zeros_like(vec)))

        iv = idx_v[...]

        copies = []
        for j in range(_J):
            col_base = base_flat + lane_scalar(iv, j)
            for r in range(rows_per_j):
                row = j * rows_per_j + r
                for q in range(q_per_row):
                    off_b = (r * q_per_row + q) * _L
                    idxbuf[row, pl.ds(q * _L, _L)] = ramp + (col_base + off_b * N)
                copies.append(
                    pltpu.async_copy(xflat.at[idxbuf.at[row]], gbuf.at[row], sem))

        wv = vals_v[...] / std_v[...]
        c = -jnp.sum(vals_v[...] * mean_v[...] / std_v[...])
        wj = [lane_scalar(wv, j) for j in range(_J)]

        for cp in copies:
            cp.wait()

        for i in range(chunks):
            r = i // q_per_row
            q = i % q_per_row
            acc = jnp.full((_L,), c, jnp.float32)
            for j in range(_J):
                acc = acc + wj[j] * gbuf[j * rows_per_j + r, pl.ds(q * _L, _L)]
            outbuf[pl.ds(i * _L, _L)] = acc

        pltpu.sync_copy(outbuf, out.at[pl.ds(base, b_per_w)])

    return sc_kernel


def kernel(X_neuron, mean, std, vals, idx):
    B, N = X_neuron.shape
    xflat = X_neuron.reshape(-1)
    pad_f = jnp.zeros((_L - _J,), jnp.float32)
    mean16 = jnp.concatenate([mean, pad_f])
    std16 = jnp.concatenate([std, jnp.ones((_L - _J,), jnp.float32)])
    vals16 = jnp.concatenate([vals, pad_f])
    idx16 = jnp.concatenate([idx, jnp.zeros((_L - _J,), jnp.int32)])
    return _make_sc_kernel(B, N)(xflat, mean16, std16, vals16, idx16)


# speedup vs baseline: 6.7957x; 6.7957x over previous
import functools

import jax
import jax.numpy as jnp
from jax import lax
from jax.experimental import pallas as pl
from jax.experimental.pallas import tpu as pltpu
from jax.experimental.pallas import tpu_sc as plsc

_NC = 2
_NS = 16
_NW = _NC * _NS
_L = 16
_J = 10
_BC = 128
_TW = 128
_NBUF = 4


@functools.cache
def _make_sc_kernel(B, N):
    b_per_w = B // _NW
    n_chunks = b_per_w // _BC
    grp_per_chunk = _BC // _L

    mesh = plsc.VectorSubcoreMesh(
        core_axis_name="c", subcore_axis_name="s",
        num_cores=_NC, num_subcores=_NS)

    @functools.partial(
        pl.kernel,
        out_type=jax.ShapeDtypeStruct((B,), jnp.float32),
        mesh=mesh,
        compiler_params=pltpu.CompilerParams(needs_layout_passes=False),
        scratch_types=[
            pltpu.VMEM((_L,), jnp.float32),
            pltpu.VMEM((_L,), jnp.float32),
            pltpu.VMEM((_L,), jnp.float32),
            pltpu.VMEM((_L,), jnp.int32),
            pltpu.VMEM((_NBUF, _BC, _TW), jnp.float32),
            pltpu.VMEM((b_per_w,), jnp.float32),
            pltpu.SemaphoreType.DMA,
            pltpu.SemaphoreType.DMA,
            pltpu.SemaphoreType.DMA,
            pltpu.SemaphoreType.DMA,
        ],
    )
    def sc_kernel(x2d, mean16, std16, vals16, idx16, out,
                  mean_v, std_v, vals_v, idx_v, ring, outbuf,
                  sem0, sem1, sem2, sem3):
        sems = (sem0, sem1, sem2, sem3)
        wid = lax.axis_index("s") * _NC + lax.axis_index("c")
        base = wid * b_per_w

        pltpu.sync_copy(mean16, mean_v)
        pltpu.sync_copy(std16, std_v)
        pltpu.sync_copy(vals16, vals_v)
        pltpu.sync_copy(idx16, idx_v)

        lane = lax.iota(jnp.int32, _L)

        def lane_scalar(vec, j):
            return jnp.sum(jnp.where(lane == j, vec, jnp.zeros_like(vec)))

        iv = idx_v[...]
        col = [lane_scalar(iv, j) for j in range(_J)]
        band = [(col[j] // _TW) * _TW for j in range(_J)]
        coff = [jnp.full((_L,), col[j] % _TW) for j in range(_J)]

        wv = vals_v[...] / std_v[...]
        c = -jnp.sum(vals_v[...] * mean_v[...] / std_v[...])
        wj = [lane_scalar(wv, j) for j in range(_J)]

        work = [(j, ch) for j in range(_J) for ch in range(n_chunks)]

        def fire(k):
            j, ch = work[k]
            slot = k % _NBUF
            return pltpu.async_copy(
                x2d.at[pl.ds(base + ch * _BC, _BC), pl.ds(band[j], _TW)],
                ring.at[slot], sems[slot])

        copies = {}
        for k in range(_NBUF):
            copies[k] = fire(k)

        for ch in range(n_chunks):
            for g in range(grp_per_chunk):
                outbuf[pl.ds(ch * _BC + g * _L, _L)] = jnp.full((_L,), c,
                                                               jnp.float32)

        for k in range(len(work)):
            j, ch = work[k]
            slot = k % _NBUF
            copies[k].wait()
            for g in range(grp_per_chunk):
                rows = jnp.full((_L,), g * _L) + lane
                vec = plsc.load_gather(ring.at[slot], [rows, coff[j]])
                off = ch * _BC + g * _L
                outbuf[pl.ds(off, _L)] = outbuf[pl.ds(off, _L)] + wj[j] * vec
            nxt = k + _NBUF
            if nxt < len(work):
                copies[nxt] = fire(nxt)

        pltpu.sync_copy(outbuf, out.at[pl.ds(base, b_per_w)])

    return sc_kernel


def kernel(X_neuron, mean, std, vals, idx):
    B, N = X_neuron.shape
    pad_f = jnp.zeros((_L - _J,), jnp.float32)
    mean16 = jnp.concatenate([mean, pad_f])
    std16 = jnp.concatenate([std, jnp.ones((_L - _J,), jnp.float32)])
    vals16 = jnp.concatenate([vals, pad_f])
    idx16 = jnp.concatenate([idx, jnp.zeros((_L - _J,), jnp.int32)])
    return _make_sc_kernel(B, N)(X_neuron, mean16, std16, vals16, idx16)


# speedup vs baseline: 7.2837x; 1.0718x over previous
import functools

import jax
import jax.numpy as jnp
from jax import lax
from jax.experimental import pallas as pl
from jax.experimental.pallas import tpu as pltpu
from jax.experimental.pallas import tpu_sc as plsc

_NC = 2
_NS = 16
_NW = _NC * _NS
_L = 16
_J = 10
_BC = 256
_TW = 128
_NBUF = 3


@functools.cache
def _make_sc_kernel(B, N):
    b_per_w = B // _NW
    n_chunks = b_per_w // _BC
    grp_per_chunk = _BC // _L

    mesh = plsc.VectorSubcoreMesh(
        core_axis_name="c", subcore_axis_name="s",
        num_cores=_NC, num_subcores=_NS)

    @functools.partial(
        pl.kernel,
        out_type=jax.ShapeDtypeStruct((B,), jnp.float32),
        mesh=mesh,
        compiler_params=pltpu.CompilerParams(needs_layout_passes=False),
        scratch_types=[
            pltpu.VMEM((4 * _L,), jnp.int32),
            pltpu.VMEM((_NBUF, _BC, _TW), jnp.float32),
            pltpu.VMEM((b_per_w,), jnp.float32),
            pltpu.SemaphoreType.DMA,
            pltpu.SemaphoreType.DMA,
            pltpu.SemaphoreType.DMA,
        ],
    )
    def sc_kernel(x2d, params, out, params_v, ring, outbuf, sem0, sem1, sem2):
        sems = (sem0, sem1, sem2)
        wid = lax.axis_index("s") * _NC + lax.axis_index("c")
        base = wid * b_per_w

        pltpu.sync_copy(params, params_v)

        lane = lax.iota(jnp.int32, _L)

        def lane_scalar(vec, j):
            return jnp.sum(jnp.where(lane == j, vec, jnp.zeros_like(vec)))

        mean_v = plsc.bitcast(params_v[pl.ds(0, _L)], jnp.float32)
        std_v = plsc.bitcast(params_v[pl.ds(_L, _L)], jnp.float32)
        vals_v = plsc.bitcast(params_v[pl.ds(2 * _L, _L)], jnp.float32)
        iv = params_v[pl.ds(3 * _L, _L)]

        col = [lane_scalar(iv, j) for j in range(_J)]
        band = [(col[j] // _TW) * _TW for j in range(_J)]
        coff = [jnp.full((_L,), col[j] % _TW) for j in range(_J)]

        wv = vals_v / std_v
        c = -jnp.sum(vals_v * mean_v / std_v)
        wj = [lane_scalar(wv, j) for j in range(_J)]

        work = [(ch, j) for ch in range(n_chunks) for j in range(_J)]

        def fire(k):
            ch, j = work[k]
            slot = k % _NBUF
            return pltpu.async_copy(
                x2d.at[pl.ds(base + ch * _BC, _BC), pl.ds(band[j], _TW)],
                ring.at[slot], sems[slot])

        copies = {}
        for k in range(_NBUF):
            copies[k] = fire(k)

        rows = [jnp.full((_L,), g * _L) + lane for g in range(grp_per_chunk)]
        for ch in range(n_chunks):
            acc = [jnp.full((_L,), c, jnp.float32)
                   for _ in range(grp_per_chunk)]
            for j in range(_J):
                k = ch * _J + j
                slot = k % _NBUF
                copies[k].wait()
                for g in range(grp_per_chunk):
                    vec = plsc.load_gather(ring.at[slot], [rows[g], coff[j]])
                    acc[g] = acc[g] + wj[j] * vec
                nxt = k + _NBUF
                if nxt < len(work):
                    copies[nxt] = fire(nxt)
            for g in range(grp_per_chunk):
                outbuf[pl.ds(ch * _BC + g * _L, _L)] = acc[g]

        pltpu.sync_copy(outbuf, out.at[pl.ds(base, b_per_w)])

    return sc_kernel


def kernel(X_neuron, mean, std, vals, idx):
    B, N = X_neuron.shape
    pad_f = jnp.zeros((_L - _J,), jnp.float32)
    mean16 = jnp.concatenate([mean, pad_f])
    std16 = jnp.concatenate([std, jnp.ones((_L - _J,), jnp.float32)])
    vals16 = jnp.concatenate([vals, pad_f])
    idx16 = jnp.concatenate([idx, jnp.zeros((_L - _J,), jnp.int32)])
    packed = jnp.concatenate([
        jax.lax.bitcast_convert_type(mean16, jnp.int32),
        jax.lax.bitcast_convert_type(std16, jnp.int32),
        jax.lax.bitcast_convert_type(vals16, jnp.int32),
        idx16,
    ])
    return _make_sc_kernel(B, N)(X_neuron, packed)


# speedup vs baseline: 7.4035x; 1.0164x over previous
import functools

import jax
import jax.numpy as jnp
from jax import lax
from jax.experimental import pallas as pl
from jax.experimental.pallas import tpu as pltpu
from jax.experimental.pallas import tpu_sc as plsc

_NC = 2
_NS = 16
_NW = _NC * _NS
_L = 16
_J = 10
_BC = 256
_TW = 128
_NBUF = 3


@functools.cache
def _make_sc_kernel(B, N):
    b_per_w = B // _NW
    n_chunks = b_per_w // _BC
    grp_per_chunk = _BC // _L

    mesh = plsc.VectorSubcoreMesh(
        core_axis_name="c", subcore_axis_name="s",
        num_cores=_NC, num_subcores=_NS)

    @functools.partial(
        pl.kernel,
        out_type=jax.ShapeDtypeStruct((B,), jnp.float32),
        mesh=mesh,
        compiler_params=pltpu.CompilerParams(needs_layout_passes=False),
        scratch_types=[
            pltpu.VMEM((_L,), jnp.float32),
            pltpu.VMEM((_L,), jnp.float32),
            pltpu.VMEM((_L,), jnp.float32),
            pltpu.VMEM((_L,), jnp.int32),
            pltpu.VMEM((_NBUF, _BC, _TW), jnp.float32),
            pltpu.VMEM((b_per_w,), jnp.float32),
            pltpu.SemaphoreType.DMA,
            pltpu.SemaphoreType.DMA,
            pltpu.SemaphoreType.DMA,
        ],
    )
    def sc_kernel(x2d, mean_in, std_in, vals_in, idx_in, out,
                  mean_v, std_v, vals_v, idx_v, ring, outbuf,
                  sem0, sem1, sem2):
        sems = (sem0, sem1, sem2)
        wid = lax.axis_index("s") * _NC + lax.axis_index("c")
        base = wid * b_per_w

        cps = [
            pltpu.async_copy(mean_in, mean_v.at[pl.ds(0, _J)], sem0),
            pltpu.async_copy(std_in, std_v.at[pl.ds(0, _J)], sem0),
            pltpu.async_copy(vals_in, vals_v.at[pl.ds(0, _J)], sem0),
            pltpu.async_copy(idx_in, idx_v.at[pl.ds(0, _J)], sem0),
        ]
        for cp in cps:
            cp.wait()

        lane = lax.iota(jnp.int32, _L)
        inb = lane < _J

        def lane_scalar(vec, j):
            return jnp.sum(jnp.where(lane == j, vec, jnp.zeros_like(vec)))

        mean_v16 = mean_v[...]
        std_v16 = std_v[...]
        vals_v16 = vals_v[...]
        iv = idx_v[...]

        col = [lane_scalar(iv, j) for j in range(_J)]
        band = [(col[j] // _TW) * _TW for j in range(_J)]
        coff = [jnp.full((_L,), col[j] % _TW) for j in range(_J)]

        wv = vals_v16 / std_v16
        c = -jnp.sum(jnp.where(inb, vals_v16 * mean_v16 / std_v16, 0.0))
        wj = [lane_scalar(wv, j) for j in range(_J)]

        work = [(ch, j) for ch in range(n_chunks) for j in range(_J)]

        def fire(k):
            ch, j = work[k]
            slot = k % _NBUF
            return pltpu.async_copy(
                x2d.at[pl.ds(base + ch * _BC, _BC), pl.ds(band[j], _TW)],
                ring.at[slot], sems[slot])

        copies = {}
        for k in range(_NBUF):
            copies[k] = fire(k)

        rows = [jnp.full((_L,), g * _L) + lane for g in range(grp_per_chunk)]
        for ch in range(n_chunks):
            acc = [jnp.full((_L,), c, jnp.float32)
                   for _ in range(grp_per_chunk)]
            for j in range(_J):
                k = ch * _J + j
                slot = k % _NBUF
                copies[k].wait()
                for g in range(grp_per_chunk):
                    vec = plsc.load_gather(ring.at[slot], [rows[g], coff[j]])
                    acc[g] = acc[g] + wj[j] * vec
                nxt = k + _NBUF
                if nxt < len(work):
                    copies[nxt] = fire(nxt)
            for g in range(grp_per_chunk):
                outbuf[pl.ds(ch * _BC + g * _L, _L)] = acc[g]

        pltpu.sync_copy(outbuf, out.at[pl.ds(base, b_per_w)])

    return sc_kernel


def kernel(X_neuron, mean, std, vals, idx):
    B, N = X_neuron.shape
    return _make_sc_kernel(B, N)(X_neuron, mean, std, vals, idx)
